# baseline (speedup 1.0000x reference)
import jax
import jax.numpy as jnp
from jax import lax
from jax.experimental import pallas as pl
from jax.experimental.pallas import tpu as pltpu
from jax.experimental.pallas import tpu_sc as plsc

N = 10000
E = 320000
D = 128
DH = D // 2
NC = 2
NS = 16
NW = NC * NS
E2 = 327680
EPT = E2 // NS
K = 64
NCHUNK = EPT // K
NPAIR = NCHUNK // 2
NP = N + 16
RPT = 624
TAIL = NP - NS * RPT
TOFF = NS * RPT


def _make_agg(with_deg):
    mesh = plsc.VectorSubcoreMesh(core_axis_name="c", subcore_axis_name="s",
                                  num_cores=NC, num_subcores=NS)
    out_type = [jax.ShapeDtypeStruct((NC * NP, DH), jnp.float32)]
    if with_deg:
        out_type.append(jax.ShapeDtypeStruct((NW, NP), jnp.float32))
    NB = 8
    scratch = [
        pltpu.VMEM((NCHUNK, K), jnp.int32),
        pltpu.VMEM((NCHUNK, K), jnp.int32),
        pltpu.VMEM_SHARED((NP, DH), jnp.float32),
    ]
    scratch += [pltpu.VMEM((K, DH), jnp.float32) for _ in range(NB)]
    scratch += [pltpu.SemaphoreType.DMA for _ in range(NB)]
    scratch += [pltpu.SemaphoreType.DMA for _ in range(NB)]
    if with_deg:
        scratch += [pltpu.VMEM((NP,), jnp.float32)]

    def body(*refs):
        if with_deg:
            (table, srcq, dstq, z2d, z1d, acc_out, deg_out,
             src_v, dst_v, acc_sh, *rest) = refs
            deg_v = rest[-1]
            rest = rest[:-1]
        else:
            (table, srcq, dstq, z2d, acc_out,
             src_v, dst_v, acc_sh, *rest) = refs
        rows = rest[:NB]
        sem_g = rest[NB:2 * NB]
        sem_s = rest[2 * NB:3 * NB]
        c = lax.axis_index("c")
        s = lax.axis_index("s")
        wid = c * NS + s

        pltpu.sync_copy(z2d, acc_sh.at[pl.ds(s * RPT, RPT)])

        @pl.when(s == 0)
        def _zero_tail():
            pltpu.sync_copy(z2d.at[pl.ds(0, TAIL)], acc_sh.at[pl.ds(TOFF, TAIL)])

        pltpu.sync_copy(srcq.at[c, s], src_v)
        pltpu.sync_copy(dstq.at[s], dst_v)
        if with_deg:
            pltpu.sync_copy(z1d, deg_v)
        ones = jnp.full((16,), 1.0, jnp.float32)

        def gath(j, b):
            return pltpu.make_async_copy(table.at[src_v.at[j]], rows[b], sem_g[b])

        def scat_start(j, b):
            pltpu.async_copy(rows[b], acc_sh.at[dst_v.at[j]], sem_s[b], add=True)

        def scat_wait(j, b):
            pltpu.make_async_copy(rows[b], acc_sh.at[dst_v.at[j]], sem_s[b]).wait()

        def count_deg(j):
            if with_deg:
                for t in range(K // 16):
                    idx = dst_v[j, pl.ds(t * 16, 16)]
                    plsc.addupdate_scatter(deg_v, [idx], ones)

        for b in range(NB):
            gath(b, b).start()
        plsc.subcore_barrier()

        NQ = NCHUNK // NB

        def quad(q, carry):
            j0 = NB * q
            for b in range(NB):
                gath(j0 + b, b).wait()
                count_deg(j0 + b)
                scat_start(j0 + b, b)

            @pl.when(q + 1 < NQ)
            def _():
                for b in range(NB):
                    scat_wait(j0 + b, b)
                    gath(j0 + NB + b, b).start()
            return carry
        lax.fori_loop(0, NQ, quad, 0)
        for b in range(NB):
            scat_wait(NCHUNK - NB + b, b)

        if with_deg:
            pltpu.sync_copy(deg_v, deg_out.at[wid])

        plsc.subcore_barrier()
        pltpu.sync_copy(acc_sh.at[pl.ds(s * RPT, RPT)],
                        acc_out.at[pl.ds(c * NP + s * RPT, RPT)])

        @pl.when(s == 0)
        def _write_tail():
            pltpu.sync_copy(acc_sh.at[pl.ds(TOFF, TAIL)],
                            acc_out.at[pl.ds(c * NP + TOFF, TAIL)])

    return pl.kernel(body, out_type=tuple(out_type), mesh=mesh,
                     scratch_types=tuple(scratch),
                     compiler_params=pltpu.CompilerParams(needs_layout_passes=False,
                                                          use_tc_tiling_on_sc=False))


_agg_deg = _make_agg(True)
_agg = _make_agg(False)

BLK = 1000


def _tc1_body(x_ref, acc_ref, degp_ref, ws_ref, wn_ref, b_ref,
              h_ref, hs_ref, dinv_ref):
    deg = 0.5 * jnp.sum(degp_ref[...], axis=1)
    dinv = 1.0 / jnp.maximum(deg, 1.0)
    agg = jnp.concatenate([acc_ref[0], acc_ref[1]], axis=1)
    hn = agg * dinv[:, None]
    h = (jnp.dot(x_ref[...], ws_ref[...], preferred_element_type=jnp.float32)
         + jnp.dot(hn, wn_ref[...], preferred_element_type=jnp.float32)
         + b_ref[...])
    h = jnp.maximum(h, 0.0)
    h_ref[...] = h
    hs_ref[0] = h[:, :DH]
    hs_ref[1] = h[:, DH:]
    dinv_ref[...] = dinv[:, None]


def _tc2_body(h_ref, acc_ref, dinv_ref, ws_ref, wn_ref, b_ref, out_ref):
    agg = jnp.concatenate([acc_ref[0], acc_ref[1]], axis=1)
    hn = agg * dinv_ref[...]
    out_ref[...] = (jnp.dot(h_ref[...], ws_ref[...], preferred_element_type=jnp.float32)
                    + jnp.dot(hn, wn_ref[...], preferred_element_type=jnp.float32)
                    + b_ref[...])


def _tc1(x, acc, degp_t, ws, wn, b):
    grid = (N // BLK,)
    return pl.pallas_call(
        _tc1_body,
        grid=grid,
        in_specs=[
            pl.BlockSpec((BLK, D), lambda i: (i, 0)),
            pl.BlockSpec((NC, BLK, DH), lambda i: (0, i, 0)),
            pl.BlockSpec((BLK, NW), lambda i: (i, 0)),
            pl.BlockSpec((D, D), lambda i: (0, 0)),
            pl.BlockSpec((D, D), lambda i: (0, 0)),
            pl.BlockSpec((1, D), lambda i: (0, 0)),
        ],
        out_specs=[
            pl.BlockSpec((BLK, D), lambda i: (i, 0)),
            pl.BlockSpec((NC, BLK, DH), lambda i: (0, i, 0)),
            pl.BlockSpec((BLK, 1), lambda i: (i, 0)),
        ],
        out_shape=[
            jax.ShapeDtypeStruct((N, D), jnp.float32),
            jax.ShapeDtypeStruct((NC, N, DH), jnp.float32),
            jax.ShapeDtypeStruct((N, 1), jnp.float32),
        ],
    )(x, acc, degp_t, ws, wn, b)


def _tc2(h, acc, dinv, ws, wn, b):
    grid = (N // BLK,)
    return pl.pallas_call(
        _tc2_body,
        grid=grid,
        in_specs=[
            pl.BlockSpec((BLK, D), lambda i: (i, 0)),
            pl.BlockSpec((NC, BLK, DH), lambda i: (0, i, 0)),
            pl.BlockSpec((BLK, 1), lambda i: (i, 0)),
            pl.BlockSpec((D, D), lambda i: (0, 0)),
            pl.BlockSpec((D, D), lambda i: (0, 0)),
            pl.BlockSpec((1, D), lambda i: (0, 0)),
        ],
        out_specs=pl.BlockSpec((BLK, D), lambda i: (i, 0)),
        out_shape=jax.ShapeDtypeStruct((N, D), jnp.float32),
    )(h, acc, dinv, ws, wn, b)


def kernel(x, edge_index, W_self1, W_neigh1, b1, W_self2, W_neigh2, b2):
    src = edge_index[0].astype(jnp.int32)
    dst = edge_index[1].astype(jnp.int32)
    pad = E2 - E
    srcp = jnp.concatenate([src, jnp.zeros((pad,), jnp.int32)]).reshape(NS, NCHUNK, K)
    pad_dst = N + (jnp.arange(pad, dtype=jnp.int32) % 16)
    dstq = jnp.concatenate([dst, pad_dst]).reshape(NS, NCHUNK, K)
    srcq = jnp.stack([srcp, srcp + N])
    z2d = jnp.zeros((RPT, DH), jnp.float32)
    z1d = jnp.zeros((NP,), jnp.float32)

    x2 = jnp.concatenate([x[:, :DH], x[:, DH:]], axis=0)
    acc1, degp = _agg_deg(x2, srcq, dstq, z2d, z1d)
    h1, h1s, dinv = _tc1(x, acc1.reshape(NC, NP, DH), degp.T,
                         W_self1.T, W_neigh1.T, b1.reshape(1, D))
    acc2, = _agg(h1s.reshape(NC * N, DH), srcq, dstq, z2d)
    out = _tc2(h1, acc2.reshape(NC, NP, DH), dinv,
               W_self2.T, W_neigh2.T, b2.reshape(1, D))
    return out


# speedup vs baseline: 1.7111x; 1.7111x over previous
import jax
import jax.numpy as jnp
from jax import lax
from jax.experimental import pallas as pl
from jax.experimental.pallas import tpu as pltpu
from jax.experimental.pallas import tpu_sc as plsc

N = 10000
E = 320000
D = 128
DH = D // 2
NC = 2
NS = 16
NW = NC * NS
E2 = 327680
EPT = E2 // NS
K = 128
NCHUNK = EPT // K
NPAIR = NCHUNK // 2
NP = N + 16
RPT = 624
TAIL = NP - NS * RPT
TOFF = NS * RPT


def _make_agg(with_deg):
    mesh = plsc.VectorSubcoreMesh(core_axis_name="c", subcore_axis_name="s",
                                  num_cores=NC, num_subcores=NS)
    out_type = [jax.ShapeDtypeStruct((NC * NP, DH), jnp.bfloat16)]
    if with_deg:
        out_type.append(jax.ShapeDtypeStruct((NW, NP), jnp.float32))
    NB = 4
    scratch = [
        pltpu.VMEM((NCHUNK, K), jnp.int32),
        pltpu.VMEM((NCHUNK, K), jnp.int32),
        pltpu.VMEM_SHARED((NP, DH), jnp.bfloat16),
    ]
    scratch += [pltpu.VMEM((K, DH), jnp.bfloat16) for _ in range(NB)]
    scratch += [pltpu.SemaphoreType.DMA for _ in range(NB)]
    scratch += [pltpu.SemaphoreType.DMA for _ in range(NB)]
    if with_deg:
        scratch += [pltpu.VMEM((NP,), jnp.float32)]

    def body(*refs):
        if with_deg:
            (table, srcq, dstq, z2d, z1d, acc_out, deg_out,
             src_v, dst_v, acc_sh, *rest) = refs
            deg_v = rest[-1]
            rest = rest[:-1]
        else:
            (table, srcq, dstq, z2d, acc_out,
             src_v, dst_v, acc_sh, *rest) = refs
        rows = rest[:NB]
        sem_g = rest[NB:2 * NB]
        sem_s = rest[2 * NB:3 * NB]
        c = lax.axis_index("c")
        s = lax.axis_index("s")
        wid = c * NS + s

        pltpu.sync_copy(z2d, acc_sh.at[pl.ds(s * RPT, RPT)])

        @pl.when(s == 0)
        def _zero_tail():
            pltpu.sync_copy(z2d.at[pl.ds(0, TAIL)], acc_sh.at[pl.ds(TOFF, TAIL)])

        pltpu.sync_copy(srcq.at[c, s], src_v)
        pltpu.sync_copy(dstq.at[s], dst_v)
        if with_deg:
            pltpu.sync_copy(z1d, deg_v)
        ones = jnp.full((16,), 1.0, jnp.float32)

        def gath(j, b):
            return pltpu.make_async_copy(table.at[src_v.at[j]], rows[b], sem_g[b])

        def scat_start(j, b):
            pltpu.async_copy(rows[b], acc_sh.at[dst_v.at[j]], sem_s[b], add=True)

        def scat_wait(j, b):
            pltpu.make_async_copy(rows[b], acc_sh.at[dst_v.at[j]], sem_s[b]).wait()

        def count_deg(j):
            if with_deg:
                for t in range(K // 16):
                    idx = dst_v[j, pl.ds(t * 16, 16)]
                    plsc.addupdate_scatter(deg_v, [idx], ones)

        for b in range(NB):
            gath(b, b).start()
        plsc.subcore_barrier()

        NQ = NCHUNK // NB

        def quad(q, carry):
            j0 = NB * q
            for b in range(NB):
                gath(j0 + b, b).wait()
                count_deg(j0 + b)
                scat_start(j0 + b, b)

            @pl.when(q + 1 < NQ)
            def _():
                for b in range(NB):
                    scat_wait(j0 + b, b)
                    gath(j0 + NB + b, b).start()
            return carry
        lax.fori_loop(0, NQ, quad, 0)
        for b in range(NB):
            scat_wait(NCHUNK - NB + b, b)

        if with_deg:
            pltpu.sync_copy(deg_v, deg_out.at[wid])

        plsc.subcore_barrier()
        pltpu.sync_copy(acc_sh.at[pl.ds(s * RPT, RPT)],
                        acc_out.at[pl.ds(c * NP + s * RPT, RPT)])

        @pl.when(s == 0)
        def _write_tail():
            pltpu.sync_copy(acc_sh.at[pl.ds(TOFF, TAIL)],
                            acc_out.at[pl.ds(c * NP + TOFF, TAIL)])

    return pl.kernel(body, out_type=tuple(out_type), mesh=mesh,
                     scratch_types=tuple(scratch),
                     compiler_params=pltpu.CompilerParams(needs_layout_passes=False,
                                                          use_tc_tiling_on_sc=False))


_agg_deg = _make_agg(True)
_agg = _make_agg(False)

BLK = 1000


def _tc1_body(x_ref, acc_ref, degp_ref, ws_ref, wn_ref, b_ref,
              h_ref, hs_ref, dinv_ref):
    deg = 0.5 * jnp.sum(degp_ref[...], axis=1)
    dinv = 1.0 / jnp.maximum(deg, 1.0)
    agg = jnp.concatenate([acc_ref[0], acc_ref[1]], axis=1).astype(jnp.float32)
    hn = agg * dinv[:, None]
    h = (jnp.dot(x_ref[...], ws_ref[...], preferred_element_type=jnp.float32)
         + jnp.dot(hn, wn_ref[...], preferred_element_type=jnp.float32)
         + b_ref[...])
    h = jnp.maximum(h, 0.0)
    h_ref[...] = h
    hb = h.astype(jnp.bfloat16)
    hs_ref[0] = hb[:, :DH]
    hs_ref[1] = hb[:, DH:]
    dinv_ref[...] = dinv[:, None]


def _tc2_body(h_ref, acc_ref, dinv_ref, ws_ref, wn_ref, b_ref, out_ref):
    agg = jnp.concatenate([acc_ref[0], acc_ref[1]], axis=1).astype(jnp.float32)
    hn = agg * dinv_ref[...]
    out_ref[...] = (jnp.dot(h_ref[...], ws_ref[...], preferred_element_type=jnp.float32)
                    + jnp.dot(hn, wn_ref[...], preferred_element_type=jnp.float32)
                    + b_ref[...])


def _tc1(x, acc, degp_t, ws, wn, b):
    grid = (N // BLK,)
    return pl.pallas_call(
        _tc1_body,
        grid=grid,
        in_specs=[
            pl.BlockSpec((BLK, D), lambda i: (i, 0)),
            pl.BlockSpec((NC, BLK, DH), lambda i: (0, i, 0)),
            pl.BlockSpec((BLK, NW), lambda i: (i, 0)),
            pl.BlockSpec((D, D), lambda i: (0, 0)),
            pl.BlockSpec((D, D), lambda i: (0, 0)),
            pl.BlockSpec((1, D), lambda i: (0, 0)),
        ],
        out_specs=[
            pl.BlockSpec((BLK, D), lambda i: (i, 0)),
            pl.BlockSpec((NC, BLK, DH), lambda i: (0, i, 0)),
            pl.BlockSpec((BLK, 1), lambda i: (i, 0)),
        ],
        out_shape=[
            jax.ShapeDtypeStruct((N, D), jnp.float32),
            jax.ShapeDtypeStruct((NC, N, DH), jnp.bfloat16),
            jax.ShapeDtypeStruct((N, 1), jnp.float32),
        ],
    )(x, acc, degp_t, ws, wn, b)


def _tc2(h, acc, dinv, ws, wn, b):
    grid = (N // BLK,)
    return pl.pallas_call(
        _tc2_body,
        grid=grid,
        in_specs=[
            pl.BlockSpec((BLK, D), lambda i: (i, 0)),
            pl.BlockSpec((NC, BLK, DH), lambda i: (0, i, 0)),
            pl.BlockSpec((BLK, 1), lambda i: (i, 0)),
            pl.BlockSpec((D, D), lambda i: (0, 0)),
            pl.BlockSpec((D, D), lambda i: (0, 0)),
            pl.BlockSpec((1, D), lambda i: (0, 0)),
        ],
        out_specs=pl.BlockSpec((BLK, D), lambda i: (i, 0)),
        out_shape=jax.ShapeDtypeStruct((N, D), jnp.float32),
    )(h, acc, dinv, ws, wn, b)


def kernel(x, edge_index, W_self1, W_neigh1, b1, W_self2, W_neigh2, b2):
    src = edge_index[0].astype(jnp.int32)
    dst = edge_index[1].astype(jnp.int32)
    pad = E2 - E
    srcp = jnp.concatenate([src, jnp.zeros((pad,), jnp.int32)]).reshape(NS, NCHUNK, K)
    pad_dst = N + (jnp.arange(pad, dtype=jnp.int32) % 16)
    dstq = jnp.concatenate([dst, pad_dst]).reshape(NS, NCHUNK, K)
    srcq = jnp.stack([srcp, srcp + N])
    z2d = jnp.zeros((RPT, DH), jnp.bfloat16)
    z1d = jnp.zeros((NP,), jnp.float32)

    x2 = jnp.concatenate([x[:, :DH], x[:, DH:]], axis=0).astype(jnp.bfloat16)
    acc1, degp = _agg_deg(x2, srcq, dstq, z2d, z1d)
    h1, h1s, dinv = _tc1(x, acc1.reshape(NC, NP, DH), degp.T,
                         W_self1.T, W_neigh1.T, b1.reshape(1, D))
    acc2, = _agg(h1s.reshape(NC * N, DH), srcq, dstq, z2d)
    out = _tc2(h1, acc2.reshape(NC, NP, DH), dinv,
               W_self2.T, W_neigh2.T, b2.reshape(1, D))
    return out


# speedup vs baseline: 1.7669x; 1.0326x over previous
import jax
import jax.numpy as jnp
from jax import lax
from jax.experimental import pallas as pl
from jax.experimental.pallas import tpu as pltpu
from jax.experimental.pallas import tpu_sc as plsc

N = 10000
E = 320000
D = 128
DH = D // 2
NC = 2
NS = 16
NW = NC * NS
E2 = 327680
EPT = E2 // NS
K = 128
NCHUNK = EPT // K
NPAIR = NCHUNK // 2
NP = N + 16
RPT = 624
TAIL = NP - NS * RPT
TOFF = NS * RPT


def _make_agg(with_deg):
    mesh = plsc.VectorSubcoreMesh(core_axis_name="c", subcore_axis_name="s",
                                  num_cores=NC, num_subcores=NS)
    out_type = [jax.ShapeDtypeStruct((NC * NP, DH), jnp.bfloat16)]
    if with_deg:
        out_type.append(jax.ShapeDtypeStruct((NW, NP), jnp.float32))
    NB = 8
    scratch = [
        pltpu.VMEM((NCHUNK, K), jnp.int32),
        pltpu.VMEM((NCHUNK, K), jnp.int32),
        pltpu.VMEM_SHARED((NP, DH), jnp.bfloat16),
    ]
    scratch += [pltpu.VMEM((K, DH), jnp.bfloat16) for _ in range(NB)]
    scratch += [pltpu.SemaphoreType.DMA for _ in range(NB)]
    scratch += [pltpu.SemaphoreType.DMA for _ in range(NB)]
    if with_deg:
        scratch += [pltpu.VMEM((NP,), jnp.float32)]

    def body(*refs):
        if with_deg:
            (table, srcq, dstq, z2d, z1d, acc_out, deg_out,
             src_v, dst_v, acc_sh, *rest) = refs
            deg_v = rest[-1]
            rest = rest[:-1]
        else:
            (table, srcq, dstq, z2d, acc_out,
             src_v, dst_v, acc_sh, *rest) = refs
        rows = rest[:NB]
        sem_g = rest[NB:2 * NB]
        sem_s = rest[2 * NB:3 * NB]
        c = lax.axis_index("c")
        s = lax.axis_index("s")
        wid = c * NS + s

        pltpu.sync_copy(z2d, acc_sh.at[pl.ds(s * RPT, RPT)])

        @pl.when(s == 0)
        def _zero_tail():
            pltpu.sync_copy(z2d.at[pl.ds(0, TAIL)], acc_sh.at[pl.ds(TOFF, TAIL)])

        pltpu.sync_copy(srcq.at[c, s], src_v)
        pltpu.sync_copy(dstq.at[s], dst_v)
        if with_deg:
            pltpu.sync_copy(z1d, deg_v)
        ones = jnp.full((16,), 1.0, jnp.float32)

        def gath(j, b):
            return pltpu.make_async_copy(table.at[src_v.at[j]], rows[b], sem_g[b])

        def scat_start(j, b):
            pltpu.async_copy(rows[b], acc_sh.at[dst_v.at[j]], sem_s[b], add=True)

        def scat_wait(j, b):
            pltpu.make_async_copy(rows[b], acc_sh.at[dst_v.at[j]], sem_s[b]).wait()

        def count_deg(j):
            if with_deg:
                for t in range(K // 16):
                    idx = dst_v[j, pl.ds(t * 16, 16)]
                    plsc.addupdate_scatter(deg_v, [idx], ones)

        for b in range(NB):
            gath(b, b).start()
        plsc.subcore_barrier()

        NQ = NCHUNK // NB

        def quad(q, carry):
            j0 = NB * q
            for b in range(NB):
                gath(j0 + b, b).wait()
                count_deg(j0 + b)
                scat_start(j0 + b, b)

            @pl.when(q + 1 < NQ)
            def _():
                for b in range(NB):
                    scat_wait(j0 + b, b)
                    gath(j0 + NB + b, b).start()
            return carry
        lax.fori_loop(0, NQ, quad, 0)
        for b in range(NB):
            scat_wait(NCHUNK - NB + b, b)

        if with_deg:
            pltpu.sync_copy(deg_v, deg_out.at[wid])

        plsc.subcore_barrier()
        pltpu.sync_copy(acc_sh.at[pl.ds(s * RPT, RPT)],
                        acc_out.at[pl.ds(c * NP + s * RPT, RPT)])

        @pl.when(s == 0)
        def _write_tail():
            pltpu.sync_copy(acc_sh.at[pl.ds(TOFF, TAIL)],
                            acc_out.at[pl.ds(c * NP + TOFF, TAIL)])

    return pl.kernel(body, out_type=tuple(out_type), mesh=mesh,
                     scratch_types=tuple(scratch),
                     compiler_params=pltpu.CompilerParams(needs_layout_passes=False,
                                                          use_tc_tiling_on_sc=False))


_agg_deg = _make_agg(True)
_agg = _make_agg(False)

BLK = 1000


def _tc1_body(x_ref, acc_ref, degp_ref, ws_ref, wn_ref, b_ref,
              h_ref, hs_ref, dinv_ref):
    deg = 0.5 * jnp.sum(degp_ref[...], axis=1)
    dinv = 1.0 / jnp.maximum(deg, 1.0)
    agg = jnp.concatenate([acc_ref[0], acc_ref[1]], axis=1).astype(jnp.float32)
    hn = agg * dinv[:, None]
    h = (jnp.dot(x_ref[...], ws_ref[...], preferred_element_type=jnp.float32)
         + jnp.dot(hn, wn_ref[...], preferred_element_type=jnp.float32)
         + b_ref[...])
    h = jnp.maximum(h, 0.0)
    h_ref[...] = h
    hb = h.astype(jnp.bfloat16)
    hs_ref[0] = hb[:, :DH]
    hs_ref[1] = hb[:, DH:]
    dinv_ref[...] = dinv[:, None]


def _tc2_body(h_ref, acc_ref, dinv_ref, ws_ref, wn_ref, b_ref, out_ref):
    agg = jnp.concatenate([acc_ref[0], acc_ref[1]], axis=1).astype(jnp.float32)
    hn = agg * dinv_ref[...]
    out_ref[...] = (jnp.dot(h_ref[...], ws_ref[...], preferred_element_type=jnp.float32)
                    + jnp.dot(hn, wn_ref[...], preferred_element_type=jnp.float32)
                    + b_ref[...])


def _tc1(x, acc, degp_t, ws, wn, b):
    grid = (N // BLK,)
    return pl.pallas_call(
        _tc1_body,
        grid=grid,
        in_specs=[
            pl.BlockSpec((BLK, D), lambda i: (i, 0)),
            pl.BlockSpec((NC, BLK, DH), lambda i: (0, i, 0)),
            pl.BlockSpec((BLK, NW), lambda i: (i, 0)),
            pl.BlockSpec((D, D), lambda i: (0, 0)),
            pl.BlockSpec((D, D), lambda i: (0, 0)),
            pl.BlockSpec((1, D), lambda i: (0, 0)),
        ],
        out_specs=[
            pl.BlockSpec((BLK, D), lambda i: (i, 0)),
            pl.BlockSpec((NC, BLK, DH), lambda i: (0, i, 0)),
            pl.BlockSpec((BLK, 1), lambda i: (i, 0)),
        ],
        out_shape=[
            jax.ShapeDtypeStruct((N, D), jnp.float32),
            jax.ShapeDtypeStruct((NC, N, DH), jnp.bfloat16),
            jax.ShapeDtypeStruct((N, 1), jnp.float32),
        ],
    )(x, acc, degp_t, ws, wn, b)


def _tc2(h, acc, dinv, ws, wn, b):
    grid = (N // BLK,)
    return pl.pallas_call(
        _tc2_body,
        grid=grid,
        in_specs=[
            pl.BlockSpec((BLK, D), lambda i: (i, 0)),
            pl.BlockSpec((NC, BLK, DH), lambda i: (0, i, 0)),
            pl.BlockSpec((BLK, 1), lambda i: (i, 0)),
            pl.BlockSpec((D, D), lambda i: (0, 0)),
            pl.BlockSpec((D, D), lambda i: (0, 0)),
            pl.BlockSpec((1, D), lambda i: (0, 0)),
        ],
        out_specs=pl.BlockSpec((BLK, D), lambda i: (i, 0)),
        out_shape=jax.ShapeDtypeStruct((N, D), jnp.float32),
    )(h, acc, dinv, ws, wn, b)


def kernel(x, edge_index, W_self1, W_neigh1, b1, W_self2, W_neigh2, b2):
    src = edge_index[0].astype(jnp.int32)
    dst = edge_index[1].astype(jnp.int32)
    pad = E2 - E
    srcp = jnp.concatenate([src, jnp.zeros((pad,), jnp.int32)]).reshape(NS, NCHUNK, K)
    pad_dst = N + (jnp.arange(pad, dtype=jnp.int32) % 16)
    dstq = jnp.concatenate([dst, pad_dst]).reshape(NS, NCHUNK, K)
    srcq = jnp.stack([srcp, srcp + N])
    z2d = jnp.zeros((RPT, DH), jnp.bfloat16)
    z1d = jnp.zeros((NP,), jnp.float32)

    x2 = jnp.concatenate([x[:, :DH], x[:, DH:]], axis=0).astype(jnp.bfloat16)
    acc1, degp = _agg_deg(x2, srcq, dstq, z2d, z1d)
    h1, h1s, dinv = _tc1(x, acc1.reshape(NC, NP, DH), degp.T,
                         W_self1.T, W_neigh1.T, b1.reshape(1, D))
    acc2, = _agg(h1s.reshape(NC * N, DH), srcq, dstq, z2d)
    out = _tc2(h1, acc2.reshape(NC, NP, DH), dinv,
               W_self2.T, W_neigh2.T, b2.reshape(1, D))
    return out
